# baseline (speedup 1.0000x reference)
import functools

import jax
import jax.numpy as jnp
from jax import lax
from jax.experimental import pallas as pl
from jax.experimental.pallas import tpu as pltpu
from jax.experimental.pallas import tpu_sc as plsc

BATCH = 16384
EMBED_DIM = 10
NPAIR = EMBED_DIM // 2
NC = 2
NS = 16
NW = NC * NS
B_PER_W = BATCH // NW
CHUNK = 128
NCHUNK = B_PER_W // CHUNK
BN = 65536


def _detile_body(ta_ref, tb_ref, o_ref):
  a = ta_ref[...].astype(jnp.bfloat16)
  b = tb_ref[0:2, :].astype(jnp.bfloat16)
  ua = lax.bitcast_convert_type(a, jnp.uint16).astype(jnp.uint32)
  ub = lax.bitcast_convert_type(b, jnp.uint16).astype(jnp.uint32)
  ua3 = ua.reshape(4, 2, BN)
  wa = ua3[:, 0, :] | (ua3[:, 1, :] << 16)
  wb = ub[0:1, :] | (ub[1:2, :] << 16)
  w = jnp.concatenate([wa, wb], axis=0)
  o_ref[...] = w.astype(jnp.int32).reshape(-1)


def _detile(tT, nb):
  return pl.pallas_call(
      _detile_body,
      grid=(nb,),
      in_specs=[
          pl.BlockSpec((8, BN), lambda j: (0, j)),
          pl.BlockSpec((8, BN), lambda j: (1, j)),
      ],
      out_specs=pl.BlockSpec((NPAIR * BN,), lambda j: (j,)),
      out_shape=jax.ShapeDtypeStruct((nb * NPAIR * BN,), jnp.int32),
  )(tT, tT)


def _gather_body(uoffs, moffs, ufl, mfl, u_out, m_out, offu, offm, outu, outm,
                 sem):
  wid = lax.axis_index("s") * NC + lax.axis_index("c")
  base = wid * B_PER_W
  pltpu.sync_copy(uoffs.at[wid], offu)
  pltpu.sync_copy(moffs.at[wid], offm)
  copies = []
  for j in range(NCHUNK):
    sl = pl.ds(j * CHUNK, CHUNK)
    for p in range(NPAIR):
      row = j * NPAIR + p
      copies.append(pltpu.async_copy(ufl.at[offu.at[row]], outu.at[p, sl], sem))
      copies.append(pltpu.async_copy(mfl.at[offm.at[row]], outm.at[p, sl], sem))
  for cp in copies:
    cp.wait()
  for p in range(NPAIR):
    dst = pl.ds(p * BATCH + base, B_PER_W)
    pltpu.sync_copy(outu.at[p], u_out.at[dst])
    pltpu.sync_copy(outm.at[p], m_out.at[dst])


_sc_gather = functools.partial(
    pl.kernel,
    out_type=(
        jax.ShapeDtypeStruct((NPAIR * BATCH,), jnp.int32),
        jax.ShapeDtypeStruct((NPAIR * BATCH,), jnp.int32),
    ),
    mesh=plsc.VectorSubcoreMesh(core_axis_name="c", subcore_axis_name="s"),
    scratch_types=[
        pltpu.VMEM((NCHUNK * NPAIR, CHUNK), jnp.int32),
        pltpu.VMEM((NCHUNK * NPAIR, CHUNK), jnp.int32),
        pltpu.VMEM((NPAIR, B_PER_W), jnp.int32),
        pltpu.VMEM((NPAIR, B_PER_W), jnp.int32),
        pltpu.SemaphoreType.DMA,
    ],
    compiler_params=pltpu.CompilerParams(
        use_tc_tiling_on_sc=False, needs_layout_passes=False),
)(_gather_body)


def _flat_offsets(idx):
  jb = idx >> 16
  base = jb * (NPAIR * BN) + (idx & (BN - 1))
  cols = (jnp.arange(NPAIR, dtype=jnp.int32) * BN)[None, :]
  o = base[:, None] + cols
  o = o.reshape(NW, NCHUNK, CHUNK, NPAIR)
  return o.transpose(0, 1, 3, 2).reshape(NW, NCHUNK * NPAIR, CHUNK)


def _unpack(w):
  w = w.reshape(NPAIR, BATCH)
  lo = lax.bitcast_convert_type((w & 0xFFFF).astype(jnp.uint16), jnp.bfloat16)
  hi = lax.bitcast_convert_type(
      ((w >> 16) & 0xFFFF).astype(jnp.uint16), jnp.bfloat16)
  return jnp.concatenate([lo, hi], axis=0).astype(jnp.float32)


def _mlp_body(u_ref, m_ref, w1u_ref, w1m_ref, b1_ref, w2_ref, b2_ref, o_ref):
  u = _unpack(u_ref[...])
  m = _unpack(m_ref[...])
  h = (
      jnp.dot(w1u_ref[...], u, preferred_element_type=jnp.float32)
      + jnp.dot(w1m_ref[...], m, preferred_element_type=jnp.float32)
      + b1_ref[...]
  )
  h = jnp.maximum(h, 0.0)
  o_ref[...] = (
      jnp.dot(w2_ref[...], h, preferred_element_type=jnp.float32) + b2_ref[...]
  )


def _mlp(u_flat, m_flat, w1u, w1m, b1, w2, b2):
  return pl.pallas_call(
      _mlp_body,
      out_shape=jax.ShapeDtypeStruct((1, BATCH), jnp.float32),
  )(u_flat, m_flat, w1u, w1m, b1, w2, b2)


@jax.jit
def kernel(user_emb_idx, movie_emb_idx, user_table, movie_table, W1, b1, W2, b2):
  nbu = -(-user_table.shape[0] // BN)
  nbm = -(-movie_table.shape[0] // BN)
  uoffs = _flat_offsets(user_emb_idx.reshape(BATCH))
  moffs = _flat_offsets(movie_emb_idx.reshape(BATCH))
  mfl = _detile(movie_table.T, nbm)
  ufl = _detile(user_table.T, nbu)
  u_flat, m_flat = _sc_gather(uoffs, moffs, ufl, mfl)
  parity = jnp.concatenate(
      [jnp.arange(0, EMBED_DIM, 2), jnp.arange(1, EMBED_DIM, 2)])
  w1u = W1[:, :EMBED_DIM][:, parity]
  w1m = W1[:, EMBED_DIM:][:, parity]
  out = _mlp(
      u_flat,
      m_flat,
      w1u,
      w1m,
      b1.reshape(-1, 1),
      W2,
      b2.reshape(1, 1),
  )
  return out.reshape(BATCH, 1)


# speedup vs baseline: 1.1740x; 1.1740x over previous
import functools

import jax
import jax.numpy as jnp
from jax import lax
from jax.experimental import pallas as pl
from jax.experimental.pallas import tpu as pltpu
from jax.experimental.pallas import tpu_sc as plsc

BATCH = 16384
EMBED_DIM = 10
NPAIR = EMBED_DIM // 2
NC = 2
NS = 16
NW = NC * NS
B_PER_W = BATCH // NW
CHUNK = 128
NCHUNK = B_PER_W // CHUNK
BN = 131072


def _detile_body(t_ref, o_ref):
  y = t_ref[...].astype(jnp.bfloat16)
  u = lax.bitcast_convert_type(y, jnp.uint16).astype(jnp.uint32)
  u3 = u.reshape(NPAIR, 2, BN)
  w = u3[:, 0, :] | (u3[:, 1, :] << 16)
  o_ref[...] = w.astype(jnp.int32).reshape(-1)


def _detile(tT, nb):
  return pl.pallas_call(
      _detile_body,
      grid=(nb,),
      in_specs=[pl.BlockSpec((EMBED_DIM, BN), lambda j: (0, j))],
      out_specs=pl.BlockSpec((NPAIR * BN,), lambda j: (j,)),
      out_shape=jax.ShapeDtypeStruct((nb * NPAIR * BN,), jnp.int32),
  )(tT)


def _gather_body(uoffs, moffs, ufl, mfl, u_out, m_out, offu, offm, outu, outm,
                 sem):
  wid = lax.axis_index("s") * NC + lax.axis_index("c")
  base = wid * B_PER_W
  pltpu.sync_copy(uoffs.at[wid], offu)
  pltpu.sync_copy(moffs.at[wid], offm)
  copies = []
  for j in range(NCHUNK):
    sl = pl.ds(j * CHUNK, CHUNK)
    for p in range(NPAIR):
      row = j * NPAIR + p
      copies.append(pltpu.async_copy(ufl.at[offu.at[row]], outu.at[p, sl], sem))
      copies.append(pltpu.async_copy(mfl.at[offm.at[row]], outm.at[p, sl], sem))
  for cp in copies:
    cp.wait()
  for p in range(NPAIR):
    dst = pl.ds(p * BATCH + base, B_PER_W)
    pltpu.sync_copy(outu.at[p], u_out.at[dst])
    pltpu.sync_copy(outm.at[p], m_out.at[dst])


_sc_gather = functools.partial(
    pl.kernel,
    out_type=(
        jax.ShapeDtypeStruct((NPAIR * BATCH,), jnp.int32),
        jax.ShapeDtypeStruct((NPAIR * BATCH,), jnp.int32),
    ),
    mesh=plsc.VectorSubcoreMesh(core_axis_name="c", subcore_axis_name="s"),
    scratch_types=[
        pltpu.VMEM((NCHUNK * NPAIR, CHUNK), jnp.int32),
        pltpu.VMEM((NCHUNK * NPAIR, CHUNK), jnp.int32),
        pltpu.VMEM((NPAIR, B_PER_W), jnp.int32),
        pltpu.VMEM((NPAIR, B_PER_W), jnp.int32),
        pltpu.SemaphoreType.DMA,
    ],
    compiler_params=pltpu.CompilerParams(
        use_tc_tiling_on_sc=False, needs_layout_passes=False),
)(_gather_body)


def _flat_offsets(idx):
  jb = idx >> 17
  base = jb * (NPAIR * BN) + (idx & (BN - 1))
  cols = (jnp.arange(NPAIR, dtype=jnp.int32) * BN)[None, :]
  o = base[:, None] + cols
  o = o.reshape(NW, NCHUNK, CHUNK, NPAIR)
  return o.transpose(0, 1, 3, 2).reshape(NW, NCHUNK * NPAIR, CHUNK)


def _unpack(w):
  w = w.reshape(NPAIR, BATCH)
  lo = lax.bitcast_convert_type((w & 0xFFFF).astype(jnp.uint16), jnp.bfloat16)
  hi = lax.bitcast_convert_type(
      ((w >> 16) & 0xFFFF).astype(jnp.uint16), jnp.bfloat16)
  return jnp.concatenate([lo, hi], axis=0).astype(jnp.float32)


def _mlp_body(u_ref, m_ref, w1u_ref, w1m_ref, b1_ref, w2_ref, b2_ref, o_ref):
  u = _unpack(u_ref[...])
  m = _unpack(m_ref[...])
  h = (
      jnp.dot(w1u_ref[...], u, preferred_element_type=jnp.float32)
      + jnp.dot(w1m_ref[...], m, preferred_element_type=jnp.float32)
      + b1_ref[...]
  )
  h = jnp.maximum(h, 0.0)
  o_ref[...] = (
      jnp.dot(w2_ref[...], h, preferred_element_type=jnp.float32) + b2_ref[...]
  )


def _mlp(u_flat, m_flat, w1u, w1m, b1, w2, b2):
  return pl.pallas_call(
      _mlp_body,
      out_shape=jax.ShapeDtypeStruct((1, BATCH), jnp.float32),
  )(u_flat, m_flat, w1u, w1m, b1, w2, b2)


@jax.jit
def kernel(user_emb_idx, movie_emb_idx, user_table, movie_table, W1, b1, W2, b2):
  nbu = -(-user_table.shape[0] // BN)
  nbm = -(-movie_table.shape[0] // BN)
  uoffs = _flat_offsets(user_emb_idx.reshape(BATCH))
  moffs = _flat_offsets(movie_emb_idx.reshape(BATCH))
  mfl = _detile(movie_table.T, nbm)
  ufl = _detile(user_table.T, nbu)
  u_flat, m_flat = _sc_gather(uoffs, moffs, ufl, mfl)
  parity = jnp.concatenate(
      [jnp.arange(0, EMBED_DIM, 2), jnp.arange(1, EMBED_DIM, 2)])
  w1u = W1[:, :EMBED_DIM][:, parity]
  w1m = W1[:, EMBED_DIM:][:, parity]
  out = _mlp(
      u_flat,
      m_flat,
      w1u,
      w1m,
      b1.reshape(-1, 1),
      W2,
      b2.reshape(1, 1),
  )
  return out.reshape(BATCH, 1)


# speedup vs baseline: 1.1826x; 1.0073x over previous
import functools

import jax
import jax.numpy as jnp
from jax import lax
from jax.experimental import pallas as pl
from jax.experimental.pallas import tpu as pltpu
from jax.experimental.pallas import tpu_sc as plsc

BATCH = 16384
EMBED_DIM = 10
NPAIR = EMBED_DIM // 2
NC = 2
NS = 16
NW = NC * NS
B_PER_W = BATCH // NW
CHUNK = 256
NCHUNK = B_PER_W // CHUNK
BN = 131072


def _detile_body(t_ref, o_ref):
  y = t_ref[...].astype(jnp.bfloat16)
  u = lax.bitcast_convert_type(y, jnp.uint16).astype(jnp.uint32)
  u3 = u.reshape(NPAIR, 2, BN)
  w = u3[:, 0, :] | (u3[:, 1, :] << 16)
  o_ref[...] = w.astype(jnp.int32).reshape(-1)


def _detile(tT, nb):
  return pl.pallas_call(
      _detile_body,
      grid=(nb,),
      in_specs=[pl.BlockSpec((EMBED_DIM, BN), lambda j: (0, j))],
      out_specs=pl.BlockSpec((NPAIR * BN,), lambda j: (j,)),
      out_shape=jax.ShapeDtypeStruct((nb * NPAIR * BN,), jnp.int32),
  )(tT)


def _gather_body(uoffs, moffs, ufl, mfl, u_out, m_out, offu, offm, outu, outm,
                 sem):
  wid = lax.axis_index("s") * NC + lax.axis_index("c")
  base = wid * B_PER_W
  pltpu.sync_copy(uoffs.at[wid], offu)
  pltpu.sync_copy(moffs.at[wid], offm)
  copies = []
  for j in range(NCHUNK):
    sl = pl.ds(j * CHUNK, CHUNK)
    for p in range(NPAIR):
      row = j * NPAIR + p
      copies.append(pltpu.async_copy(ufl.at[offu.at[row]], outu.at[p, sl], sem))
      copies.append(pltpu.async_copy(mfl.at[offm.at[row]], outm.at[p, sl], sem))
  for cp in copies:
    cp.wait()
  for p in range(NPAIR):
    dst = pl.ds(p * BATCH + base, B_PER_W)
    pltpu.sync_copy(outu.at[p], u_out.at[dst])
    pltpu.sync_copy(outm.at[p], m_out.at[dst])


_sc_gather = functools.partial(
    pl.kernel,
    out_type=(
        jax.ShapeDtypeStruct((NPAIR * BATCH,), jnp.int32),
        jax.ShapeDtypeStruct((NPAIR * BATCH,), jnp.int32),
    ),
    mesh=plsc.VectorSubcoreMesh(core_axis_name="c", subcore_axis_name="s"),
    scratch_types=[
        pltpu.VMEM((NCHUNK * NPAIR, CHUNK), jnp.int32),
        pltpu.VMEM((NCHUNK * NPAIR, CHUNK), jnp.int32),
        pltpu.VMEM((NPAIR, B_PER_W), jnp.int32),
        pltpu.VMEM((NPAIR, B_PER_W), jnp.int32),
        pltpu.SemaphoreType.DMA,
    ],
    compiler_params=pltpu.CompilerParams(
        use_tc_tiling_on_sc=False, needs_layout_passes=False),
)(_gather_body)


def _flat_offsets(idx):
  jb = idx >> 17
  base = jb * (NPAIR * BN) + (idx & (BN - 1))
  cols = (jnp.arange(NPAIR, dtype=jnp.int32) * BN)[None, :]
  o = base[:, None] + cols
  o = o.reshape(NW, NCHUNK, CHUNK, NPAIR)
  return o.transpose(0, 1, 3, 2).reshape(NW, NCHUNK * NPAIR, CHUNK)


def _unpack(w):
  w = w.reshape(NPAIR, BATCH)
  lo = lax.bitcast_convert_type((w & 0xFFFF).astype(jnp.uint16), jnp.bfloat16)
  hi = lax.bitcast_convert_type(
      ((w >> 16) & 0xFFFF).astype(jnp.uint16), jnp.bfloat16)
  return jnp.concatenate([lo, hi], axis=0).astype(jnp.float32)


def _mlp_body(u_ref, m_ref, w1u_ref, w1m_ref, b1_ref, w2_ref, b2_ref, o_ref):
  u = _unpack(u_ref[...])
  m = _unpack(m_ref[...])
  h = (
      jnp.dot(w1u_ref[...], u, preferred_element_type=jnp.float32)
      + jnp.dot(w1m_ref[...], m, preferred_element_type=jnp.float32)
      + b1_ref[...]
  )
  h = jnp.maximum(h, 0.0)
  o_ref[...] = (
      jnp.dot(w2_ref[...], h, preferred_element_type=jnp.float32) + b2_ref[...]
  )


def _mlp(u_flat, m_flat, w1u, w1m, b1, w2, b2):
  return pl.pallas_call(
      _mlp_body,
      out_shape=jax.ShapeDtypeStruct((1, BATCH), jnp.float32),
  )(u_flat, m_flat, w1u, w1m, b1, w2, b2)


@jax.jit
def kernel(user_emb_idx, movie_emb_idx, user_table, movie_table, W1, b1, W2, b2):
  nbu = -(-user_table.shape[0] // BN)
  nbm = -(-movie_table.shape[0] // BN)
  uoffs = _flat_offsets(user_emb_idx.reshape(BATCH))
  moffs = _flat_offsets(movie_emb_idx.reshape(BATCH))
  mfl = _detile(movie_table.T, nbm)
  ufl = _detile(user_table.T, nbu)
  u_flat, m_flat = _sc_gather(uoffs, moffs, ufl, mfl)
  parity = jnp.concatenate(
      [jnp.arange(0, EMBED_DIM, 2), jnp.arange(1, EMBED_DIM, 2)])
  w1u = W1[:, :EMBED_DIM][:, parity]
  w1m = W1[:, EMBED_DIM:][:, parity]
  out = _mlp(
      u_flat,
      m_flat,
      w1u,
      w1m,
      b1.reshape(-1, 1),
      W2,
      b2.reshape(1, 1),
  )
  return out.reshape(BATCH, 1)


# speedup vs baseline: 1.2447x; 1.0525x over previous
import functools

import jax
import jax.numpy as jnp
from jax import lax
from jax.experimental import pallas as pl
from jax.experimental.pallas import tpu as pltpu
from jax.experimental.pallas import tpu_sc as plsc

BATCH = 16384
EMBED_DIM = 10
NPAIR = EMBED_DIM // 2
NC = 2
NS = 16
NW = NC * NS
B_PER_W = BATCH // NW
CHUNK = 256
NCHUNK = B_PER_W // CHUNK
BN = 131072


def _detile_body(t_ref, o_ref):
  y = t_ref[...].astype(jnp.bfloat16)
  u = lax.bitcast_convert_type(y, jnp.uint16).astype(jnp.uint32)
  u3 = u.reshape(NPAIR, 2, BN)
  w = u3[:, 0, :] | (u3[:, 1, :] << 16)
  o_ref[...] = w.astype(jnp.int32).reshape(-1)


def _detile(tT, nb):
  return pl.pallas_call(
      _detile_body,
      grid=(nb,),
      in_specs=[pl.BlockSpec((EMBED_DIM, BN), lambda j: (0, j))],
      out_specs=pl.BlockSpec((NPAIR * BN,), lambda j: (j,)),
      out_shape=jax.ShapeDtypeStruct((nb * NPAIR * BN,), jnp.int32),
  )(tT)


def _compute_offsets(idx, offbuf):
  for s in range(B_PER_W // 16):
    sl16 = pl.ds(s * 16, 16)
    r = idx[sl16]
    jb = r >> 17
    base = (jb << 19) + (jb << 17) + (r & (BN - 1))
    j, l = divmod(s * 16, CHUNK)
    for p in range(NPAIR):
      offbuf[j * NPAIR + p, pl.ds(l, 16)] = base + p * BN


def _gather_body(u_idx, m_idx, ufl, mfl, u_out, m_out, idxu, idxm, offu, offm,
                 outu, outm, sem):
  wid = lax.axis_index("s") * NC + lax.axis_index("c")
  base = wid * B_PER_W
  pltpu.sync_copy(u_idx.at[wid], idxu)
  pltpu.sync_copy(m_idx.at[wid], idxm)
  _compute_offsets(idxu, offu)
  _compute_offsets(idxm, offm)
  copies = []
  for j in range(NCHUNK):
    sl = pl.ds(j * CHUNK, CHUNK)
    for p in range(NPAIR):
      row = j * NPAIR + p
      copies.append(pltpu.async_copy(ufl.at[offu.at[row]], outu.at[p, sl], sem))
      copies.append(pltpu.async_copy(mfl.at[offm.at[row]], outm.at[p, sl], sem))
  for cp in copies:
    cp.wait()
  for p in range(NPAIR):
    dst = pl.ds(p * BATCH + base, B_PER_W)
    pltpu.sync_copy(outu.at[p], u_out.at[dst])
    pltpu.sync_copy(outm.at[p], m_out.at[dst])


_sc_gather = functools.partial(
    pl.kernel,
    out_type=(
        jax.ShapeDtypeStruct((NPAIR * BATCH,), jnp.int32),
        jax.ShapeDtypeStruct((NPAIR * BATCH,), jnp.int32),
    ),
    mesh=plsc.VectorSubcoreMesh(core_axis_name="c", subcore_axis_name="s"),
    scratch_types=[
        pltpu.VMEM((B_PER_W,), jnp.int32),
        pltpu.VMEM((B_PER_W,), jnp.int32),
        pltpu.VMEM((NCHUNK * NPAIR, CHUNK), jnp.int32),
        pltpu.VMEM((NCHUNK * NPAIR, CHUNK), jnp.int32),
        pltpu.VMEM((NPAIR, B_PER_W), jnp.int32),
        pltpu.VMEM((NPAIR, B_PER_W), jnp.int32),
        pltpu.SemaphoreType.DMA,
    ],
    compiler_params=pltpu.CompilerParams(
        use_tc_tiling_on_sc=False, needs_layout_passes=False),
)(_gather_body)


def _unpack(w):
  w = w.reshape(NPAIR, BATCH)
  lo = lax.bitcast_convert_type((w & 0xFFFF).astype(jnp.uint16), jnp.bfloat16)
  hi = lax.bitcast_convert_type(
      ((w >> 16) & 0xFFFF).astype(jnp.uint16), jnp.bfloat16)
  return jnp.concatenate([lo, hi], axis=0)


def _mlp_body(u_ref, m_ref, w1u_ref, w1m_ref, b1_ref, w2_ref, b2_ref, o_ref):
  u = _unpack(u_ref[...])
  m = _unpack(m_ref[...])
  w1u = w1u_ref[...].astype(jnp.bfloat16)
  w1m = w1m_ref[...].astype(jnp.bfloat16)
  h = (
      jnp.dot(w1u, u, preferred_element_type=jnp.float32)
      + jnp.dot(w1m, m, preferred_element_type=jnp.float32)
      + b1_ref[...]
  )
  h = jnp.maximum(h, 0.0).astype(jnp.bfloat16)
  w2 = w2_ref[...].astype(jnp.bfloat16)
  o_ref[...] = (
      jnp.dot(w2, h, preferred_element_type=jnp.float32) + b2_ref[...]
  )


def _mlp(u_flat, m_flat, w1u, w1m, b1, w2, b2):
  return pl.pallas_call(
      _mlp_body,
      out_shape=jax.ShapeDtypeStruct((1, BATCH), jnp.float32),
  )(u_flat, m_flat, w1u, w1m, b1, w2, b2)


@jax.jit
def kernel(user_emb_idx, movie_emb_idx, user_table, movie_table, W1, b1, W2, b2):
  nbu = -(-user_table.shape[0] // BN)
  nbm = -(-movie_table.shape[0] // BN)
  u_idx = user_emb_idx.reshape(NW, B_PER_W)
  m_idx = movie_emb_idx.reshape(NW, B_PER_W)
  mfl = _detile(movie_table.T, nbm)
  ufl = _detile(user_table.T, nbu)
  u_flat, m_flat = _sc_gather(u_idx, m_idx, ufl, mfl)
  parity = jnp.concatenate(
      [jnp.arange(0, EMBED_DIM, 2), jnp.arange(1, EMBED_DIM, 2)])
  w1u = W1[:, :EMBED_DIM][:, parity]
  w1m = W1[:, EMBED_DIM:][:, parity]
  out = _mlp(
      u_flat,
      m_flat,
      w1u,
      w1m,
      b1.reshape(-1, 1),
      W2,
      b2.reshape(1, 1),
  )
  return out.reshape(BATCH, 1)


# speedup vs baseline: 1.2454x; 1.0005x over previous
import functools

import jax
import jax.numpy as jnp
from jax import lax
from jax.experimental import pallas as pl
from jax.experimental.pallas import tpu as pltpu
from jax.experimental.pallas import tpu_sc as plsc

BATCH = 16384
EMBED_DIM = 10
NPAIR = EMBED_DIM // 2
NC = 2
NS = 16
NW = NC * NS
B_PER_W = BATCH // NW
CHUNK = 512
NCHUNK = B_PER_W // CHUNK
BN = 131072


def _detile_body(t_ref, o_ref):
  y = t_ref[...].astype(jnp.bfloat16)
  u = lax.bitcast_convert_type(y, jnp.uint16).astype(jnp.uint32)
  u3 = u.reshape(NPAIR, 2, BN)
  w = u3[:, 0, :] | (u3[:, 1, :] << 16)
  o_ref[...] = w.astype(jnp.int32).reshape(-1)


def _detile(tT, nb):
  return pl.pallas_call(
      _detile_body,
      grid=(nb,),
      in_specs=[pl.BlockSpec((EMBED_DIM, BN), lambda j: (0, j))],
      out_specs=pl.BlockSpec((NPAIR * BN,), lambda j: (j,)),
      out_shape=jax.ShapeDtypeStruct((nb * NPAIR * BN,), jnp.int32),
  )(tT)


def _compute_offsets(idx, offbuf):
  for s in range(B_PER_W // 16):
    sl16 = pl.ds(s * 16, 16)
    r = idx[sl16]
    jb = r >> 17
    base = (jb << 19) + (jb << 17) + (r & (BN - 1))
    j, l = divmod(s * 16, CHUNK)
    for p in range(NPAIR):
      offbuf[j * NPAIR + p, pl.ds(l, 16)] = base + p * BN


def _gather_body(u_idx, m_idx, ufl, mfl, u_out, m_out, idxu, idxm, offu, offm,
                 outu, outm, sem):
  wid = lax.axis_index("s") * NC + lax.axis_index("c")
  base = wid * B_PER_W
  pltpu.sync_copy(u_idx.at[wid], idxu)
  pltpu.sync_copy(m_idx.at[wid], idxm)
  _compute_offsets(idxu, offu)
  _compute_offsets(idxm, offm)
  copies = []
  for j in range(NCHUNK):
    sl = pl.ds(j * CHUNK, CHUNK)
    for p in range(NPAIR):
      row = j * NPAIR + p
      copies.append(pltpu.async_copy(ufl.at[offu.at[row]], outu.at[p, sl], sem))
      copies.append(pltpu.async_copy(mfl.at[offm.at[row]], outm.at[p, sl], sem))
  for cp in copies:
    cp.wait()
  for p in range(NPAIR):
    dst = pl.ds(p * BATCH + base, B_PER_W)
    pltpu.sync_copy(outu.at[p], u_out.at[dst])
    pltpu.sync_copy(outm.at[p], m_out.at[dst])


_sc_gather = functools.partial(
    pl.kernel,
    out_type=(
        jax.ShapeDtypeStruct((NPAIR * BATCH,), jnp.int32),
        jax.ShapeDtypeStruct((NPAIR * BATCH,), jnp.int32),
    ),
    mesh=plsc.VectorSubcoreMesh(core_axis_name="c", subcore_axis_name="s"),
    scratch_types=[
        pltpu.VMEM((B_PER_W,), jnp.int32),
        pltpu.VMEM((B_PER_W,), jnp.int32),
        pltpu.VMEM((NCHUNK * NPAIR, CHUNK), jnp.int32),
        pltpu.VMEM((NCHUNK * NPAIR, CHUNK), jnp.int32),
        pltpu.VMEM((NPAIR, B_PER_W), jnp.int32),
        pltpu.VMEM((NPAIR, B_PER_W), jnp.int32),
        pltpu.SemaphoreType.DMA,
    ],
    compiler_params=pltpu.CompilerParams(
        use_tc_tiling_on_sc=False, needs_layout_passes=False),
)(_gather_body)


def _unpack(w):
  w = w.reshape(NPAIR, BATCH)
  lo = lax.bitcast_convert_type((w & 0xFFFF).astype(jnp.uint16), jnp.bfloat16)
  hi = lax.bitcast_convert_type(
      ((w >> 16) & 0xFFFF).astype(jnp.uint16), jnp.bfloat16)
  return jnp.concatenate([lo, hi], axis=0)


def _mlp_body(u_ref, m_ref, w1u_ref, w1m_ref, b1_ref, w2_ref, b2_ref, o_ref):
  u = _unpack(u_ref[...])
  m = _unpack(m_ref[...])
  w1u = w1u_ref[...].astype(jnp.bfloat16)
  w1m = w1m_ref[...].astype(jnp.bfloat16)
  h = (
      jnp.dot(w1u, u, preferred_element_type=jnp.float32)
      + jnp.dot(w1m, m, preferred_element_type=jnp.float32)
      + b1_ref[...]
  )
  h = jnp.maximum(h, 0.0).astype(jnp.bfloat16)
  w2 = w2_ref[...].astype(jnp.bfloat16)
  o_ref[...] = (
      jnp.dot(w2, h, preferred_element_type=jnp.float32) + b2_ref[...]
  )


def _mlp(u_flat, m_flat, w1u, w1m, b1, w2, b2):
  return pl.pallas_call(
      _mlp_body,
      out_shape=jax.ShapeDtypeStruct((1, BATCH), jnp.float32),
  )(u_flat, m_flat, w1u, w1m, b1, w2, b2)


@jax.jit
def kernel(user_emb_idx, movie_emb_idx, user_table, movie_table, W1, b1, W2, b2):
  nbu = -(-user_table.shape[0] // BN)
  nbm = -(-movie_table.shape[0] // BN)
  u_idx = user_emb_idx.reshape(NW, B_PER_W)
  m_idx = movie_emb_idx.reshape(NW, B_PER_W)
  mfl = _detile(movie_table.T, nbm)
  ufl = _detile(user_table.T, nbu)
  u_flat, m_flat = _sc_gather(u_idx, m_idx, ufl, mfl)
  parity = jnp.concatenate(
      [jnp.arange(0, EMBED_DIM, 2), jnp.arange(1, EMBED_DIM, 2)])
  w1u = W1[:, :EMBED_DIM][:, parity]
  w1m = W1[:, EMBED_DIM:][:, parity]
  out = _mlp(
      u_flat,
      m_flat,
      w1u,
      w1m,
      b1.reshape(-1, 1),
      W2,
      b2.reshape(1, 1),
  )
  return out.reshape(BATCH, 1)
